# baseline (device time: 21916 ns/iter reference)
import jax
import jax.numpy as jnp
from jax import lax
from jax.experimental import pallas as pl
from jax.experimental.pallas import tpu as pltpu

N_DEV = 32
QUAD_MASKS = (1, 2, 3)
MID_MASKS = (4, 8, 12, 16, 20, 24, 28)
N_MID = len(MID_MASKS)
N_HALF = 2


def kernel(x):
    m, n = x.shape
    hrows = m // N_HALF
    qrows = hrows // 4

    def body(x_ref, out_ref, acc, rs_recv, mid_recv, ag_recv,
             rs_send_sems, rs_recv_sems, mid_send_sems, mid_recv_sems,
             ag_send_sems, ag_recv_sems):
        my_pos = lax.axis_index("i")
        j_me = my_pos % 4

        barrier_sem = pltpu.get_barrier_semaphore()
        for mask in QUAD_MASKS + MID_MASKS:
            pl.semaphore_signal(
                barrier_sem, inc=1,
                device_id=(my_pos ^ mask,),
                device_id_type=pl.DeviceIdType.MESH,
            )
        pl.semaphore_wait(barrier_sem, len(QUAD_MASKS) + N_MID)

        done = []

        def start(rdma):
            rdma.start()
            done.append(rdma)
            return rdma

        rs = {}
        for h in range(N_HALF):
            for k, mask in enumerate(QUAD_MASKS):
                j_dst = j_me ^ mask
                rs[(h, k)] = start(pltpu.make_async_remote_copy(
                    src_ref=x_ref.at[pl.ds(h * hrows + j_dst * qrows, qrows), :],
                    dst_ref=rs_recv.at[h, k],
                    send_sem=rs_send_sems.at[h, k],
                    recv_sem=rs_recv_sems.at[h, k],
                    device_id=(my_pos ^ mask,),
                    device_id_type=pl.DeviceIdType.MESH,
                ))

        def rs_finish(h):
            for k in range(3):
                rs[(h, k)].wait_recv()
            acc[h, 0] = (
                x_ref[pl.ds(h * hrows + j_me * qrows, qrows), :]
                + rs_recv[h, 0] + rs_recv[h, 1] + rs_recv[h, 2]
            )

        mid = {}

        def mid_start(h):
            for k, mask in enumerate(MID_MASKS):
                mid[(h, k)] = start(pltpu.make_async_remote_copy(
                    src_ref=acc.at[h, 0],
                    dst_ref=mid_recv.at[h, k],
                    send_sem=mid_send_sems.at[h, k],
                    recv_sem=mid_recv_sems.at[h, k],
                    device_id=(my_pos ^ mask,),
                    device_id_type=pl.DeviceIdType.MESH,
                ))

        def mid_finish(h):
            for k in range(N_MID):
                mid[(h, k)].wait_recv()
            acc[h, 1] = (
                acc[h, 0]
                + mid_recv[h, 0] + mid_recv[h, 1] + mid_recv[h, 2]
                + mid_recv[h, 3] + mid_recv[h, 4] + mid_recv[h, 5]
                + mid_recv[h, 6]
            )

        ag = {}

        def ag_start(h):
            out_ref[pl.ds(h * hrows + j_me * qrows, qrows), :] = acc[h, 1]
            for k, mask in enumerate(QUAD_MASKS):
                ag[(h, k)] = start(pltpu.make_async_remote_copy(
                    src_ref=acc.at[h, 1],
                    dst_ref=ag_recv.at[h, k],
                    send_sem=ag_send_sems.at[h, k],
                    recv_sem=ag_recv_sems.at[h, k],
                    device_id=(my_pos ^ mask,),
                    device_id_type=pl.DeviceIdType.MESH,
                ))

        def ag_finish(h):
            for k, mask in enumerate(QUAD_MASKS):
                ag[(h, k)].wait_recv()
                out_ref[pl.ds(h * hrows + (j_me ^ mask) * qrows, qrows), :] = (
                    ag_recv[h, k]
                )

        rs_finish(0)
        mid_start(0)
        rs_finish(1)
        mid_start(1)
        mid_finish(0)
        ag_start(0)
        mid_finish(1)
        ag_start(1)
        ag_finish(0)
        ag_finish(1)

        for rdma in done:
            rdma.wait_send()

    return pl.pallas_call(
        body,
        out_shape=jax.ShapeDtypeStruct((m, n), x.dtype),
        in_specs=[pl.BlockSpec(memory_space=pltpu.VMEM)],
        out_specs=pl.BlockSpec(memory_space=pltpu.VMEM),
        scratch_shapes=[
            pltpu.VMEM((N_HALF, 2, qrows, n), x.dtype),
            pltpu.VMEM((N_HALF, 3, qrows, n), x.dtype),
            pltpu.VMEM((N_HALF, N_MID, qrows, n), x.dtype),
            pltpu.VMEM((N_HALF, 3, qrows, n), x.dtype),
            pltpu.SemaphoreType.DMA((N_HALF, 3)),
            pltpu.SemaphoreType.DMA((N_HALF, 3)),
            pltpu.SemaphoreType.DMA((N_HALF, N_MID)),
            pltpu.SemaphoreType.DMA((N_HALF, N_MID)),
            pltpu.SemaphoreType.DMA((N_HALF, 3)),
            pltpu.SemaphoreType.DMA((N_HALF, 3)),
        ],
        compiler_params=pltpu.CompilerParams(collective_id=0),
    )(x)


# device time: 20070 ns/iter; 1.0920x vs baseline; 1.0920x over previous
import jax
import jax.numpy as jnp
from jax import lax
from jax.experimental import pallas as pl
from jax.experimental.pallas import tpu as pltpu

N_DEV = 32
QUAD_MASKS = (1, 2, 3)
Z_MASKS = (8, 16, 24)
Y2_MASK = 4
ALL_PEER_MASKS = (1, 2, 3, 4, 8, 16, 24)
N_HALF = 2


def kernel(x):
    m, n = x.shape
    hrows = m // N_HALF
    qrows = hrows // 4

    def body(x_ref, out_ref, acc, rs_recv, z_recv, y_recv,
             rs_send_sems, rs_recv_sems, z_send_sems, z_recv_sems,
             y_send_sems, y_recv_sems, ag_send_sems, ag_recv_sems):
        my_pos = lax.axis_index("i")
        j_me = my_pos % 4

        barrier_sem = pltpu.get_barrier_semaphore()
        for mask in ALL_PEER_MASKS:
            pl.semaphore_signal(
                barrier_sem, inc=1,
                device_id=(my_pos ^ mask,),
                device_id_type=pl.DeviceIdType.MESH,
            )
        pl.semaphore_wait(barrier_sem, len(ALL_PEER_MASKS))

        done = []

        def start(rdma):
            rdma.start()
            done.append(rdma)
            return rdma

        rs = {}
        for h in range(N_HALF):
            for k, mask in enumerate(QUAD_MASKS):
                j_dst = j_me ^ mask
                rs[(h, k)] = start(pltpu.make_async_remote_copy(
                    src_ref=x_ref.at[pl.ds(h * hrows + j_dst * qrows, qrows), :],
                    dst_ref=rs_recv.at[h, k],
                    send_sem=rs_send_sems.at[h, k],
                    recv_sem=rs_recv_sems.at[h, k],
                    device_id=(my_pos ^ mask,),
                    device_id_type=pl.DeviceIdType.MESH,
                ))

        def rs_finish(h):
            for k in range(3):
                rs[(h, k)].wait_recv()
            acc[h, 0] = (
                x_ref[pl.ds(h * hrows + j_me * qrows, qrows), :]
                + rs_recv[h, 0] + rs_recv[h, 1] + rs_recv[h, 2]
            )

        z = {}

        def z_start(h, a):
            for k, mask in enumerate(Z_MASKS):
                z[(h, k)] = start(pltpu.make_async_remote_copy(
                    src_ref=acc.at[h, a],
                    dst_ref=z_recv.at[h, k],
                    send_sem=z_send_sems.at[h, k],
                    recv_sem=z_recv_sems.at[h, k],
                    device_id=(my_pos ^ mask,),
                    device_id_type=pl.DeviceIdType.MESH,
                ))

        def z_finish(h, a):
            for k in range(3):
                z[(h, k)].wait_recv()
            acc[h, a + 1] = (
                acc[h, a] + z_recv[h, 0] + z_recv[h, 1] + z_recv[h, 2]
            )

        y = {}

        def y_start(h, a):
            y[h] = start(pltpu.make_async_remote_copy(
                src_ref=acc.at[h, a],
                dst_ref=y_recv.at[h],
                send_sem=y_send_sems.at[h],
                recv_sem=y_recv_sems.at[h],
                device_id=(my_pos ^ Y2_MASK,),
                device_id_type=pl.DeviceIdType.MESH,
            ))

        def y_finish(h, a):
            y[h].wait_recv()
            acc[h, a + 1] = acc[h, a] + y_recv[h]

        ag = {}

        def ag_start(h):
            out_ref[pl.ds(h * hrows + j_me * qrows, qrows), :] = acc[h, 2]
            for k, mask in enumerate(QUAD_MASKS):
                ag[(h, k)] = start(pltpu.make_async_remote_copy(
                    src_ref=out_ref.at[pl.ds(h * hrows + j_me * qrows, qrows), :],
                    dst_ref=out_ref.at[pl.ds(h * hrows + j_me * qrows, qrows), :],
                    send_sem=ag_send_sems.at[h, k],
                    recv_sem=ag_recv_sems.at[h, k],
                    device_id=(my_pos ^ mask,),
                    device_id_type=pl.DeviceIdType.MESH,
                ))

        def ag_finish(h):
            for k in range(3):
                ag[(h, k)].wait_recv()

        rs_finish(0)
        z_start(0, 0)
        rs_finish(1)
        y_start(1, 0)
        z_finish(0, 0)
        y_start(0, 1)
        y_finish(1, 0)
        z_start(1, 1)
        y_finish(0, 1)
        ag_start(0)
        z_finish(1, 1)
        ag_start(1)
        ag_finish(0)
        ag_finish(1)

        for rdma in done:
            rdma.wait_send()

    return pl.pallas_call(
        body,
        out_shape=jax.ShapeDtypeStruct((m, n), x.dtype),
        in_specs=[pl.BlockSpec(memory_space=pltpu.VMEM)],
        out_specs=pl.BlockSpec(memory_space=pltpu.VMEM),
        scratch_shapes=[
            pltpu.VMEM((N_HALF, 3, qrows, n), x.dtype),
            pltpu.VMEM((N_HALF, 3, qrows, n), x.dtype),
            pltpu.VMEM((N_HALF, 3, qrows, n), x.dtype),
            pltpu.VMEM((N_HALF, qrows, n), x.dtype),
            pltpu.SemaphoreType.DMA((N_HALF, 3)),
            pltpu.SemaphoreType.DMA((N_HALF, 3)),
            pltpu.SemaphoreType.DMA((N_HALF, 3)),
            pltpu.SemaphoreType.DMA((N_HALF, 3)),
            pltpu.SemaphoreType.DMA((N_HALF,)),
            pltpu.SemaphoreType.DMA((N_HALF,)),
            pltpu.SemaphoreType.DMA((N_HALF, 3)),
            pltpu.SemaphoreType.DMA((N_HALF, 3)),
        ],
        compiler_params=pltpu.CompilerParams(collective_id=0),
    )(x)
